# initial kernel scaffold (unmeasured)
import jax
import jax.numpy as jnp
from jax import lax
from jax.experimental import pallas as pl
from jax.experimental.pallas import tpu as pltpu

N_DEV = 8
HQ_LOC = 8
DH = 128
SQ = 1024
SKV_LOC = 1024
QBLK = 64
SCALE = 0.08838834764831843
NEG = -1e9


def _body(x_ref, wq_ref, wo_ref, ksend_hbm, vsend_hbm, out_ref,
          kgath, vgath, q_ref, acc_ref, m_ref, l_ref, rsbuf,
          loc_sem, ks_send, ks_recv, vs_send, vs_recv, rs_send, rs_recv):
    me = lax.axis_index("i")

    bar = pltpu.get_barrier_semaphore()
    for o in range(1, N_DEV):
        pl.semaphore_signal(bar, inc=1, device_id=((me + o) % N_DEV,),
                            device_id_type=pl.DeviceIdType.MESH)
    pl.semaphore_wait(bar, N_DEV - 1)

    kloc = pltpu.make_async_copy(ksend_hbm.at[0], kgath.at[0], loc_sem.at[0])
    vloc = pltpu.make_async_copy(vsend_hbm.at[0], vgath.at[0], loc_sem.at[1])
    kloc.start()
    vloc.start()

    k_rdmas, v_rdmas = [], []
    for o in range(1, N_DEV):
        dest = (me + o) % N_DEV
        slot = N_DEV - o
        kr = pltpu.make_async_remote_copy(
            src_ref=ksend_hbm.at[o], dst_ref=kgath.at[slot],
            send_sem=ks_send.at[o - 1], recv_sem=ks_recv.at[slot - 1],
            device_id=(dest,), device_id_type=pl.DeviceIdType.MESH)
        kr.start()
        vr = pltpu.make_async_remote_copy(
            src_ref=vsend_hbm.at[o], dst_ref=vgath.at[slot],
            send_sem=vs_send.at[o - 1], recv_sem=vs_recv.at[slot - 1],
            device_id=(dest,), device_id_type=pl.DeviceIdType.MESH)
        vr.start()
        k_rdmas.append(kr)
        v_rdmas.append(vr)

    qf = jnp.dot(x_ref[...], wq_ref[...], preferred_element_type=jnp.float32)
    for h in range(HQ_LOC):
        q_ref[h] = qf[:, h * DH:(h + 1) * DH].astype(jnp.bfloat16)
    acc_ref[...] = jnp.zeros_like(acc_ref)
    m_ref[...] = jnp.full_like(m_ref, -1e30)
    l_ref[...] = jnp.zeros_like(l_ref)

    kloc.wait()
    vloc.wait()
    for r in k_rdmas:
        r.wait_recv()
    for r in v_rdmas:
        r.wait_recv()

    def chunk_step(s, carry):
        j = (me + s) % N_DEV
        qb = lax.broadcasted_iota(jnp.int32, (SQ, SKV_LOC), 0) // QBLK
        kb = lax.broadcasted_iota(jnp.int32, (SQ, SKV_LOC), 1) // QBLK + j * (SKV_LOC // QBLK)
        mask = (qb == kb) | (kb == 0) | ((qb + kb) % 3 == 0)
        for h in range(HQ_LOC):
            k_ch = kgath[s, h]
            v_ch = vgath[s, h]
            sc = lax.dot_general(q_ref[h], k_ch, (((1,), (1,)), ((), ())),
                                 preferred_element_type=jnp.float32)
            sc = jnp.where(mask, sc, NEG)
            m_prev = m_ref[h]
            m_new = jnp.maximum(m_prev, jnp.max(sc, axis=1, keepdims=True))
            p = jnp.exp(sc - m_new)
            alpha = jnp.exp(m_prev - m_new)
            l_ref[h] = l_ref[h] * alpha + jnp.sum(p, axis=1, keepdims=True)
            acc_ref[h] = acc_ref[h] * alpha + lax.dot_general(
                p.astype(jnp.bfloat16), v_ch, (((1,), (0,)), ((), ())),
                preferred_element_type=jnp.float32)
            m_ref[h] = m_new
        return carry

    lax.fori_loop(0, N_DEV, chunk_step, 0)

    for h in range(HQ_LOC):
        ctx_h = (acc_ref[h] / l_ref[h]).astype(jnp.bfloat16)
        contrib = jnp.dot(ctx_h, wo_ref[h * DH:(h + 1) * DH, :],
                          preferred_element_type=jnp.float32)
        if h == 0:
            out_ref[...] = contrib
        else:
            out_ref[...] = out_ref[...] + contrib

    right = (me + 1) % N_DEV
    C = SQ // N_DEV
    rs_rdmas = []
    for t in range(N_DEV - 1):
        send_idx = (me - t) % N_DEV
        recv_idx = (me - t - 1) % N_DEV
        rd = pltpu.make_async_remote_copy(
            src_ref=out_ref.at[pl.ds(send_idx * C, C), :],
            dst_ref=rsbuf.at[t % 2],
            send_sem=rs_send.at[t], recv_sem=rs_recv.at[t],
            device_id=(right,), device_id_type=pl.DeviceIdType.MESH)
        rd.start()
        rd.wait_recv()
        out_ref[pl.ds(recv_idx * C, C), :] = (
            out_ref[pl.ds(recv_idx * C, C), :] + rsbuf[t % 2])
        rs_rdmas.append(rd)
    for t2 in range(N_DEV - 1):
        t = N_DEV - 1 + t2
        send_idx = (me + 1 - t2) % N_DEV
        recv_idx = (me - t2) % N_DEV
        rd = pltpu.make_async_remote_copy(
            src_ref=out_ref.at[pl.ds(send_idx * C, C), :],
            dst_ref=rsbuf.at[t % 2],
            send_sem=rs_send.at[t], recv_sem=rs_recv.at[t],
            device_id=(right,), device_id_type=pl.DeviceIdType.MESH)
        rd.start()
        rd.wait_recv()
        out_ref[pl.ds(recv_idx * C, C), :] = rsbuf[t % 2]
        rs_rdmas.append(rd)

    for r in k_rdmas:
        r.wait_send()
    for r in v_rdmas:
        r.wait_send()
    for r in rs_rdmas:
        r.wait_send()


def kernel(x, Wq, K_ext, V_ext, Wo):
    me = lax.axis_index("i")

    xb = x[0].astype(jnp.bfloat16)
    wq = (Wq * SCALE).astype(jnp.bfloat16)
    wo = Wo.astype(jnp.bfloat16)

    def prep(a):
        t = a[0].reshape(SKV_LOC, N_DEV, HQ_LOC, DH).transpose(1, 2, 0, 3)
        order = (me + jnp.arange(N_DEV)) % N_DEV
        return jnp.take(t, order, axis=0).astype(jnp.bfloat16)

    ksend = prep(K_ext)
    vsend = prep(V_ext)

    out2d = pl.pallas_call(
        _body,
        out_shape=jax.ShapeDtypeStruct((SQ, 1024), jnp.float32),
        in_specs=[
            pl.BlockSpec(memory_space=pltpu.VMEM),
            pl.BlockSpec(memory_space=pltpu.VMEM),
            pl.BlockSpec(memory_space=pltpu.VMEM),
            pl.BlockSpec(memory_space=pl.ANY),
            pl.BlockSpec(memory_space=pl.ANY),
        ],
        out_specs=pl.BlockSpec(memory_space=pltpu.VMEM),
        scratch_shapes=[
            pltpu.VMEM((N_DEV, HQ_LOC, SKV_LOC, DH), jnp.bfloat16),
            pltpu.VMEM((N_DEV, HQ_LOC, SKV_LOC, DH), jnp.bfloat16),
            pltpu.VMEM((HQ_LOC, SQ, DH), jnp.bfloat16),
            pltpu.VMEM((HQ_LOC, SQ, DH), jnp.float32),
            pltpu.VMEM((HQ_LOC, SQ, 1), jnp.float32),
            pltpu.VMEM((HQ_LOC, SQ, 1), jnp.float32),
            pltpu.VMEM((2, SQ // N_DEV, 1024), jnp.float32),
            pltpu.SemaphoreType.DMA((2,)),
            pltpu.SemaphoreType.DMA((N_DEV - 1,)),
            pltpu.SemaphoreType.DMA((N_DEV - 1,)),
            pltpu.SemaphoreType.DMA((N_DEV - 1,)),
            pltpu.SemaphoreType.DMA((N_DEV - 1,)),
            pltpu.SemaphoreType.DMA((2 * (N_DEV - 1),)),
            pltpu.SemaphoreType.DMA((2 * (N_DEV - 1),)),
        ],
        compiler_params=pltpu.CompilerParams(collective_id=0),
    )(xb, wq, wo, ksend, vsend)

    return out2d.reshape(1, SQ, 1024)


# baseline (device time: 704719 ns/iter reference)
import jax
import jax.numpy as jnp
from jax import lax
from jax.experimental import pallas as pl
from jax.experimental.pallas import tpu as pltpu

N_DEV = 8
HQ_LOC = 8
DH = 128
SQ = 1024
SKV_LOC = 1024
QBLK = 64
SCALE = 0.08838834764831843
NEG = -1e9


def _body(q_ref, wo_ref, ksend_hbm, vsend_hbm, out_ref,
          kgath, vgath, acc_ref, ml_ref, sc_ref, bias_ref, rsbuf,
          loc_sem, ks_send, ks_recv, vs_send, vs_recv, rs_send, rs_recv,
          credit):
    me = lax.axis_index("i")

    bar = pltpu.get_barrier_semaphore()
    for o in range(1, N_DEV):
        pl.semaphore_signal(bar, inc=1, device_id=((me + o) % N_DEV,),
                            device_id_type=pl.DeviceIdType.MESH)
    pl.semaphore_wait(bar, N_DEV - 1)

    kloc = pltpu.make_async_copy(ksend_hbm.at[0], kgath.at[0], loc_sem.at[0])
    vloc = pltpu.make_async_copy(vsend_hbm.at[0], vgath.at[0], loc_sem.at[1])
    kloc.start()
    vloc.start()

    k_rdmas, v_rdmas = [], []
    for o in range(1, N_DEV):
        dest = (me + o) % N_DEV
        slot = N_DEV - o
        kr = pltpu.make_async_remote_copy(
            src_ref=ksend_hbm.at[o], dst_ref=kgath.at[slot],
            send_sem=ks_send.at[o - 1], recv_sem=ks_recv.at[slot - 1],
            device_id=(dest,), device_id_type=pl.DeviceIdType.MESH)
        kr.start()
        vr = pltpu.make_async_remote_copy(
            src_ref=vsend_hbm.at[o], dst_ref=vgath.at[slot],
            send_sem=vs_send.at[o - 1], recv_sem=vs_recv.at[slot - 1],
            device_id=(dest,), device_id_type=pl.DeviceIdType.MESH)
        vr.start()
        k_rdmas.append(kr)
        v_rdmas.append(vr)

    acc_ref[...] = jnp.zeros_like(acc_ref)
    ml_ref[:, 0:HQ_LOC] = jnp.full((SQ, HQ_LOC), -1e30, jnp.float32)
    ml_ref[:, HQ_LOC:2 * HQ_LOC] = jnp.zeros((SQ, HQ_LOC), jnp.float32)

    kloc.wait()
    vloc.wait()
    for r in k_rdmas:
        r.wait_recv()
    for r in v_rdmas:
        r.wait_recv()

    def chunk_step(s, carry):
        j = (me + s) % N_DEV
        qb = lax.broadcasted_iota(jnp.int32, (SQ, SKV_LOC), 0) // QBLK
        kb = (lax.broadcasted_iota(jnp.int32, (SQ, SKV_LOC), 1) // QBLK
              + j * (SKV_LOC // QBLK))
        mask = (qb == kb) | (kb == 0) | ((qb + kb) % 3 == 0)
        bias_ref[...] = jnp.where(mask, 0.0, NEG).astype(jnp.bfloat16)
        for h in range(HQ_LOC):
            k_ch = kgath[s, h]
            v_ch = vgath[s, h]
            sc_ref[...] = lax.dot_general(
                q_ref[h], k_ch, (((1,), (1,)), ((), ())),
                preferred_element_type=jnp.float32) + bias_ref[...]
            m_prev = ml_ref[:, h:h + 1]
            m_new = jnp.maximum(m_prev, jnp.max(sc_ref[...], axis=1,
                                                keepdims=True))
            alpha = jnp.exp(m_prev - m_new)
            sc_ref[...] = jnp.exp(sc_ref[...] - m_new)
            ml_ref[:, HQ_LOC + h:HQ_LOC + h + 1] = (
                ml_ref[:, HQ_LOC + h:HQ_LOC + h + 1] * alpha
                + jnp.sum(sc_ref[...], axis=1, keepdims=True))
            acc_ref[h] = acc_ref[h] * alpha + lax.dot_general(
                sc_ref[...].astype(jnp.bfloat16), v_ch,
                (((1,), (0,)), ((), ())), preferred_element_type=jnp.float32)
            ml_ref[:, h:h + 1] = m_new
        return carry

    lax.fori_loop(0, N_DEV, chunk_step, 0)

    for h in range(HQ_LOC):
        ctx_h = (acc_ref[h] / ml_ref[:, HQ_LOC + h:HQ_LOC + h + 1]).astype(
            jnp.bfloat16)
        contrib = jnp.dot(ctx_h, wo_ref[h * DH:(h + 1) * DH, :],
                          preferred_element_type=jnp.float32)
        if h == 0:
            out_ref[...] = contrib
        else:
            out_ref[...] = out_ref[...] + contrib

    right = (me + 1) % N_DEV
    left = (me - 1) % N_DEV
    C = SQ // N_DEV
    nsteps = 2 * (N_DEV - 1)
    rs_rdmas = []
    for t in range(nsteps):
        if t < N_DEV - 1:
            send_idx = (me - t) % N_DEV
            recv_idx = (me - t - 1) % N_DEV
        else:
            t2 = t - (N_DEV - 1)
            send_idx = (me + 1 - t2) % N_DEV
            recv_idx = (me - t2) % N_DEV
        if t >= 2:
            pl.semaphore_wait(credit.at[t % 2], 1)
        rd = pltpu.make_async_remote_copy(
            src_ref=out_ref.at[pl.ds(send_idx * C, C), :],
            dst_ref=rsbuf.at[t % 2],
            send_sem=rs_send.at[t], recv_sem=rs_recv.at[t],
            device_id=(right,), device_id_type=pl.DeviceIdType.MESH)
        rd.start()
        rd.wait_recv()
        if t < N_DEV - 1:
            out_ref[pl.ds(recv_idx * C, C), :] = (
                out_ref[pl.ds(recv_idx * C, C), :] + rsbuf[t % 2])
        else:
            out_ref[pl.ds(recv_idx * C, C), :] = rsbuf[t % 2]
        if t < nsteps - 2:
            pl.semaphore_signal(credit.at[t % 2], inc=1, device_id=(left,),
                                device_id_type=pl.DeviceIdType.MESH)
        rs_rdmas.append(rd)

    for r in k_rdmas:
        r.wait_send()
    for r in v_rdmas:
        r.wait_send()
    for r in rs_rdmas:
        r.wait_send()


def kernel(x, Wq, K_ext, V_ext, Wo):
    me = lax.axis_index("i")

    qf = jnp.dot(x[0], Wq * SCALE, preferred_element_type=jnp.float32)
    q = qf.reshape(SQ, HQ_LOC, DH).transpose(1, 0, 2).astype(jnp.bfloat16)
    wo = Wo.astype(jnp.bfloat16)

    def prep(a):
        t = a[0].reshape(SKV_LOC, N_DEV, HQ_LOC, DH).transpose(1, 2, 0, 3)
        order = (me + jnp.arange(N_DEV)) % N_DEV
        return jnp.take(t, order, axis=0).astype(jnp.bfloat16)

    ksend = prep(K_ext)
    vsend = prep(V_ext)

    out2d = pl.pallas_call(
        _body,
        out_shape=jax.ShapeDtypeStruct((SQ, 1024), jnp.float32),
        in_specs=[
            pl.BlockSpec(memory_space=pltpu.VMEM),
            pl.BlockSpec(memory_space=pltpu.VMEM),
            pl.BlockSpec(memory_space=pl.ANY),
            pl.BlockSpec(memory_space=pl.ANY),
        ],
        out_specs=pl.BlockSpec(memory_space=pltpu.VMEM),
        scratch_shapes=[
            pltpu.VMEM((N_DEV, HQ_LOC, SKV_LOC, DH), jnp.bfloat16),
            pltpu.VMEM((N_DEV, HQ_LOC, SKV_LOC, DH), jnp.bfloat16),
            pltpu.VMEM((HQ_LOC, SQ, DH), jnp.float32),
            pltpu.VMEM((SQ, 2 * HQ_LOC), jnp.float32),
            pltpu.VMEM((SQ, SKV_LOC), jnp.float32),
            pltpu.VMEM((SQ, SKV_LOC), jnp.bfloat16),
            pltpu.VMEM((2, SQ // N_DEV, 1024), jnp.float32),
            pltpu.SemaphoreType.DMA((2,)),
            pltpu.SemaphoreType.DMA((N_DEV - 1,)),
            pltpu.SemaphoreType.DMA((N_DEV - 1,)),
            pltpu.SemaphoreType.DMA((N_DEV - 1,)),
            pltpu.SemaphoreType.DMA((N_DEV - 1,)),
            pltpu.SemaphoreType.DMA((2 * (N_DEV - 1),)),
            pltpu.SemaphoreType.DMA((2 * (N_DEV - 1),)),
            pltpu.SemaphoreType.REGULAR((2,)),
        ],
        compiler_params=pltpu.CompilerParams(
            collective_id=0, vmem_limit_bytes=64 * 1024 * 1024),
    )(q, wo, ksend, vsend)

    return out2d.reshape(1, SQ, 1024)


# device time: 538427 ns/iter; 1.3088x vs baseline; 1.3088x over previous
import jax
import jax.numpy as jnp
from jax import lax
from jax.experimental import pallas as pl
from jax.experimental.pallas import tpu as pltpu

N_DEV = 8
HQ_LOC = 8
DH = 128
SQ = 1024
SKV_LOC = 1024
QBLK = 64
SCALE = 0.08838834764831843
NEG = -1e9


def _body(q_ref, wo_ref, ksend_hbm, vsend_hbm, out_ref,
          kgath, vgath, acc_ref, ml_ref, sc_ref, bias_ref, rsbuf,
          loc_sem, ks_send, ks_recv, vs_send, vs_recv, rs_send, rs_recv,
          credit):
    me = lax.axis_index("i")

    bar = pltpu.get_barrier_semaphore()
    for o in range(1, N_DEV):
        pl.semaphore_signal(bar, inc=1, device_id=((me + o) % N_DEV,),
                            device_id_type=pl.DeviceIdType.MESH)
    pl.semaphore_wait(bar, N_DEV - 1)

    kloc = pltpu.make_async_copy(ksend_hbm.at[0], kgath.at[0], loc_sem.at[0])
    vloc = pltpu.make_async_copy(vsend_hbm.at[0], vgath.at[0], loc_sem.at[1])
    kloc.start()
    vloc.start()

    k_rdmas, v_rdmas = [], []
    for o in range(1, N_DEV):
        dest = (me + o) % N_DEV
        slot = N_DEV - o
        kr = pltpu.make_async_remote_copy(
            src_ref=ksend_hbm.at[o], dst_ref=kgath.at[slot],
            send_sem=ks_send.at[o - 1], recv_sem=ks_recv.at[slot - 1],
            device_id=(dest,), device_id_type=pl.DeviceIdType.MESH)
        kr.start()
        vr = pltpu.make_async_remote_copy(
            src_ref=vsend_hbm.at[o], dst_ref=vgath.at[slot],
            send_sem=vs_send.at[o - 1], recv_sem=vs_recv.at[slot - 1],
            device_id=(dest,), device_id_type=pl.DeviceIdType.MESH)
        vr.start()
        k_rdmas.append(kr)
        v_rdmas.append(vr)

    acc_ref[...] = jnp.zeros_like(acc_ref)
    ml_ref[:, 0:HQ_LOC] = jnp.full((SQ, HQ_LOC), -1e30, jnp.float32)
    ml_ref[:, HQ_LOC:2 * HQ_LOC] = jnp.zeros((SQ, HQ_LOC), jnp.float32)

    kloc.wait()
    vloc.wait()

    def chunk_step(t, carry):
        s = (N_DEV - t) % N_DEV
        for tt in range(1, N_DEV):
            ss = N_DEV - tt

            @pl.when(t == tt)
            def _():
                k_rdmas[7 - ss].wait_recv()
                v_rdmas[7 - ss].wait_recv()

        j = (me + s) % N_DEV
        qb = lax.broadcasted_iota(jnp.int32, (SQ, SKV_LOC), 0) // QBLK
        kb = (lax.broadcasted_iota(jnp.int32, (SQ, SKV_LOC), 1) // QBLK
              + j * (SKV_LOC // QBLK))
        mask = (qb == kb) | (kb == 0) | ((qb + kb) % 3 == 0)
        bias_ref[...] = jnp.where(mask, 0.0, NEG).astype(jnp.bfloat16)
        for h in range(HQ_LOC):
            k_ch = kgath[s, h]
            v_ch = vgath[s, h]
            sc_ref[...] = lax.dot_general(
                q_ref[h], k_ch, (((1,), (1,)), ((), ())),
                preferred_element_type=jnp.float32) + bias_ref[...]
            m_prev = ml_ref[:, h:h + 1]
            m_new = jnp.maximum(m_prev, jnp.max(sc_ref[...], axis=1,
                                                keepdims=True))
            alpha = jnp.exp(m_prev - m_new)
            sc_ref[...] = jnp.exp(sc_ref[...] - m_new)
            ml_ref[:, HQ_LOC + h:HQ_LOC + h + 1] = (
                ml_ref[:, HQ_LOC + h:HQ_LOC + h + 1] * alpha
                + jnp.sum(sc_ref[...], axis=1, keepdims=True))
            acc_ref[h] = acc_ref[h] * alpha + lax.dot_general(
                sc_ref[...].astype(jnp.bfloat16), v_ch,
                (((1,), (0,)), ((), ())), preferred_element_type=jnp.float32)
            ml_ref[:, h:h + 1] = m_new
        return carry

    lax.fori_loop(0, N_DEV, chunk_step, 0)

    for h in range(HQ_LOC):
        ctx_h = (acc_ref[h] / ml_ref[:, HQ_LOC + h:HQ_LOC + h + 1]).astype(
            jnp.bfloat16)
        contrib = jnp.dot(ctx_h, wo_ref[h * DH:(h + 1) * DH, :],
                          preferred_element_type=jnp.float32)
        if h == 0:
            out_ref[...] = contrib
        else:
            out_ref[...] = out_ref[...] + contrib

    right = (me + 1) % N_DEV
    left = (me - 1) % N_DEV
    C = SQ // N_DEV
    nsteps = 2 * (N_DEV - 1)
    rs_rdmas = []
    for t in range(nsteps):
        if t < N_DEV - 1:
            send_idx = (me - t) % N_DEV
            recv_idx = (me - t - 1) % N_DEV
        else:
            t2 = t - (N_DEV - 1)
            send_idx = (me + 1 - t2) % N_DEV
            recv_idx = (me - t2) % N_DEV
        if t >= 2:
            pl.semaphore_wait(credit.at[t % 2], 1)
        rd = pltpu.make_async_remote_copy(
            src_ref=out_ref.at[pl.ds(send_idx * C, C), :],
            dst_ref=rsbuf.at[t % 2],
            send_sem=rs_send.at[t], recv_sem=rs_recv.at[t],
            device_id=(right,), device_id_type=pl.DeviceIdType.MESH)
        rd.start()
        rd.wait_recv()
        if t < N_DEV - 1:
            out_ref[pl.ds(recv_idx * C, C), :] = (
                out_ref[pl.ds(recv_idx * C, C), :] + rsbuf[t % 2])
        else:
            out_ref[pl.ds(recv_idx * C, C), :] = rsbuf[t % 2]
        if t < nsteps - 2:
            pl.semaphore_signal(credit.at[t % 2], inc=1, device_id=(left,),
                                device_id_type=pl.DeviceIdType.MESH)
        rs_rdmas.append(rd)

    for r in k_rdmas:
        r.wait_send()
    for r in v_rdmas:
        r.wait_send()
    for r in rs_rdmas:
        r.wait_send()


def kernel(x, Wq, K_ext, V_ext, Wo):
    me = lax.axis_index("i")

    qf = jnp.dot(x[0], Wq * SCALE, preferred_element_type=jnp.float32)
    q = qf.reshape(SQ, HQ_LOC, DH).transpose(1, 0, 2).astype(jnp.bfloat16)
    wo = Wo.astype(jnp.bfloat16)

    def prep(a):
        t = a[0].reshape(SKV_LOC, N_DEV, HQ_LOC, DH).transpose(1, 2, 0, 3)
        order = (me + jnp.arange(N_DEV)) % N_DEV
        return jnp.take(t, order, axis=0).astype(jnp.bfloat16)

    ksend = prep(K_ext)
    vsend = prep(V_ext)

    out2d = pl.pallas_call(
        _body,
        out_shape=jax.ShapeDtypeStruct((SQ, 1024), jnp.float32),
        in_specs=[
            pl.BlockSpec(memory_space=pltpu.VMEM),
            pl.BlockSpec(memory_space=pltpu.VMEM),
            pl.BlockSpec(memory_space=pl.ANY),
            pl.BlockSpec(memory_space=pl.ANY),
        ],
        out_specs=pl.BlockSpec(memory_space=pltpu.VMEM),
        scratch_shapes=[
            pltpu.VMEM((N_DEV, HQ_LOC, SKV_LOC, DH), jnp.bfloat16),
            pltpu.VMEM((N_DEV, HQ_LOC, SKV_LOC, DH), jnp.bfloat16),
            pltpu.VMEM((HQ_LOC, SQ, DH), jnp.float32),
            pltpu.VMEM((SQ, 2 * HQ_LOC), jnp.float32),
            pltpu.VMEM((SQ, SKV_LOC), jnp.float32),
            pltpu.VMEM((SQ, SKV_LOC), jnp.bfloat16),
            pltpu.VMEM((2, SQ // N_DEV, 1024), jnp.float32),
            pltpu.SemaphoreType.DMA((2,)),
            pltpu.SemaphoreType.DMA((N_DEV - 1,)),
            pltpu.SemaphoreType.DMA((N_DEV - 1,)),
            pltpu.SemaphoreType.DMA((N_DEV - 1,)),
            pltpu.SemaphoreType.DMA((N_DEV - 1,)),
            pltpu.SemaphoreType.DMA((2 * (N_DEV - 1),)),
            pltpu.SemaphoreType.DMA((2 * (N_DEV - 1),)),
            pltpu.SemaphoreType.REGULAR((2,)),
        ],
        compiler_params=pltpu.CompilerParams(
            collective_id=0, vmem_limit_bytes=64 * 1024 * 1024),
    )(q, wo, ksend, vsend)

    return out2d.reshape(1, SQ, 1024)
